# baseline (speedup 1.0000x reference)
import functools

import numpy as np
import jax
import jax.numpy as jnp
from jax import lax
from jax.experimental import pallas as pl
from jax.experimental.pallas import tpu as pltpu
from jax.experimental.pallas import tpu_sc as plsc

T_STEPS = 2048
S = 32
B = 16
LANES = 16
K = 16
L_PACK = 17408
EXP_ROWS = 256
NCHUNK = 4
TN = T_STEPS // NCHUNK

_bs_static = 16 - (np.arange(T_STEPS) // 128)
_off_static = np.concatenate([[0], np.cumsum(_bs_static)]).astype(np.int64)

_chunk_row_base = [int(_off_static[c * TN]) for c in range(NCHUNK + 1)]


def _chunk_rows(c):
    base = _chunk_row_base[c]
    nrows = _chunk_row_base[c + 1] - base
    rows = np.zeros((B, TN), np.int32)
    for b in range(B):
        Lb = T_STEPS - 128 * b
        steps = min(max(Lb - c * TN, 0), TN)
        if steps:
            rows[b, :steps] = _off_static[c * TN:c * TN + steps] - base + b
    return rows, nrows


def _bf16_bits(x):
    xb = jax.lax.bitcast_convert_type(x, jnp.int32)
    t = xb + 0x7FFF + ((xb >> 16) & 1)
    return (t >> 16) & 0xFFFF


def _exp_body(x_ref, o_ref):
    x = x_ref[...]
    ea = jnp.exp(x[:, :S * LANES])
    eb = jnp.exp(x[:, S * LANES:])
    o_ref[...] = jax.lax.bitcast_convert_type(
        _bf16_bits(ea) | (_bf16_bits(eb) << 16), jnp.float32)


def _make_exp_tc(nrows):
    return pl.pallas_call(
        _exp_body,
        grid=(nrows // EXP_ROWS,),
        in_specs=[pl.BlockSpec((EXP_ROWS, S * S), lambda i: (i, 0))],
        out_specs=pl.BlockSpec((EXP_ROWS, S * LANES), lambda i: (i, 0)),
        out_shape=jax.ShapeDtypeStruct((nrows, S * LANES), jnp.float32),
    )


def _reg_bcast(vec, i):
    return jnp.take_along_axis(vec, jnp.full((LANES,), i, jnp.int32), axis=0,
                               mode="promise_in_bounds")


def _tec_body(T0, p_hbm, rows_hbm, uin_hbm, ein_hbm, uout_hbm, eout_hbm,
              idx_v, buf_v, u_v, e_v, sem0, sem1):
    info = plsc.get_sparse_core_info()
    nc = info.num_cores
    wid = lax.axis_index("s") * nc + lax.axis_index("c")

    @pl.when(wid < B)
    def _run():
        b = wid
        steps = jnp.clip(T_STEPS - 128 * b - T0, 0, TN)
        nb = steps // K

        pltpu.sync_copy(uin_hbm.at[b], u_v)
        pltpu.sync_copy(ein_hbm.at[b], e_v)

        @pl.when(nb > 0)
        def _scan():
            pltpu.sync_copy(rows_hbm.at[b], idx_v)

            def _gather(blk, slot, sem):
                return pltpu.make_async_copy(
                    p_hbm.at[idx_v.at[pl.ds(blk * K, K)]], buf_v.at[slot], sem)

            sems = (sem0, sem1)
            _gather(0, 0, sem0).start()

            def _row_step(r, slot):
                u0v = u_v[pl.ds(0, LANES)]
                u1v = u_v[pl.ds(LANES, LANES)]
                p0 = [jnp.zeros((LANES,), jnp.float32) for _ in range(4)]
                p1 = [jnp.zeros((LANES,), jnp.float32) for _ in range(4)]
                for k in range(S):
                    uhalf = u0v if k < LANES else u1v
                    uk = _reg_bcast(uhalf, k % LANES)
                    w = buf_v[slot, r, pl.ds(k * LANES, LANES)]
                    x = plsc.bitcast(w, jnp.bfloat16)
                    a0, a1 = plsc.unpack(x, format=plsc.PackFormat.INTERLEAVED)
                    p0[k % 4] = p0[k % 4] + a0 * uk
                    p1[k % 4] = p1[k % 4] + a1 * uk
                acc0 = (p0[0] + p0[1]) + (p0[2] + p0[3])
                acc1 = (p1[0] + p1[1]) + (p1[2] + p1[3])
                u_v[pl.ds(0, LANES)] = acc0
                u_v[pl.ds(LANES, LANES)] = acc1

                @pl.when((r & 3) == 3)
                def _rescale():
                    u0 = u_v[pl.ds(0, LANES)]
                    u1 = u_v[pl.ds(LANES, LANES)]
                    sv = jnp.full((LANES,), jnp.sum(u0 + u1), jnp.float32)
                    e = (plsc.bitcast(sv, jnp.int32) >> 23) - 127
                    f = plsc.bitcast((127 - e) << 23, jnp.float32)
                    u_v[pl.ds(0, LANES)] = u0 * f
                    u_v[pl.ds(LANES, LANES)] = u1 * f
                    e_v[...] = e_v[...] + e.astype(jnp.float32)

            def _pair_body(i2, carry):
                for par in (0, 1):
                    blk = 2 * i2 + par
                    @pl.when(blk + 1 < nb)
                    def _prefetch():
                        _gather(blk + 1, 1 - par, sems[1 - par]).start()
                    _gather(blk, par, sems[par]).wait()

                    def _rows_f(r, cc):
                        _row_step(r, par)
                        return cc
                    lax.fori_loop(0, K, _rows_f, 0)
                return carry

            lax.fori_loop(0, nb // 2, _pair_body, 0)

        pltpu.sync_copy(u_v, uout_hbm.at[b])
        pltpu.sync_copy(e_v, eout_hbm.at[b])


def _make_sc(T0, nrows):
    @functools.partial(
        pl.kernel,
        out_type=(jax.ShapeDtypeStruct((B, S), jnp.float32),
                  jax.ShapeDtypeStruct((B, LANES), jnp.float32)),
        mesh=plsc.VectorSubcoreMesh(core_axis_name="c", subcore_axis_name="s"),
        compiler_params=pltpu.CompilerParams(needs_layout_passes=False),
        scratch_types=[
            pltpu.VMEM((TN,), jnp.int32),
            pltpu.VMEM((2, K, S * LANES), jnp.float32),
            pltpu.VMEM((S,), jnp.float32),
            pltpu.VMEM((LANES,), jnp.float32),
            pltpu.SemaphoreType.DMA,
            pltpu.SemaphoreType.DMA,
        ],
    )
    def _sc(p_hbm, rows_hbm, uin_hbm, ein_hbm, uout_hbm, eout_hbm,
            idx_v, buf_v, u_v, e_v, sem0, sem1):
        _tec_body(T0, p_hbm, rows_hbm, uin_hbm, ein_hbm, uout_hbm, eout_hbm,
                  idx_v, buf_v, u_v, e_v, sem0, sem1)
    return _sc


@jax.jit
def kernel(theta_data, batch_sizes):
    u = jnp.ones((B, S), jnp.float32)
    e = jnp.zeros((B, LANES), jnp.float32)
    for c in range(NCHUNK):
        rows_np, nrows = _chunk_rows(c)
        base = _chunk_row_base[c]
        th = lax.slice_in_dim(theta_data, base, base + nrows, axis=0)
        th2 = jnp.transpose(
            th.reshape(nrows, 2, LANES, S), (0, 1, 3, 2)).reshape(nrows, S * S)
        p2d = _make_exp_tc(nrows)(th2)
        u, e = _make_sc(c * TN, nrows)(p2d, jnp.asarray(rows_np), u, e)
    delta = (jnp.sum(batch_sizes) - L_PACK).astype(jnp.float32)
    ln2 = jnp.float32(np.log(2.0))
    return jnp.log(jnp.sum(u, axis=1)) + e[:, 0] * ln2 + delta


# speedup vs baseline: 1.0434x; 1.0434x over previous
import functools

import numpy as np
import jax
import jax.numpy as jnp
from jax import lax
from jax.experimental import pallas as pl
from jax.experimental.pallas import tpu as pltpu
from jax.experimental.pallas import tpu_sc as plsc

T_STEPS = 2048
S = 32
B = 16
LANES = 16
K = 16
L_PACK = 17408
EXP_ROWS = 512

_bs_static = 16 - (np.arange(T_STEPS) // 128)
_off_static = np.concatenate([[0], np.cumsum(_bs_static)])
_rows_np = np.zeros((B, T_STEPS), np.int32)
for _b in range(B):
    _Lb = T_STEPS - 128 * _b
    _rows_np[_b, :_Lb] = _off_static[:_Lb] + _b


def _bf16_bits(x):
    xb = jax.lax.bitcast_convert_type(x, jnp.int32)
    t = xb + 0x7FFF + ((xb >> 16) & 1)
    return (t >> 16) & 0xFFFF


def _exp_body(x_ref, o_ref):
    x = x_ref[...]
    ea = jnp.exp(x[:, :S * LANES])
    eb = jnp.exp(x[:, S * LANES:])
    o_ref[...] = jax.lax.bitcast_convert_type(
        _bf16_bits(ea) | (_bf16_bits(eb) << 16), jnp.float32)


_exp_tc = pl.pallas_call(
    _exp_body,
    grid=(L_PACK // EXP_ROWS,),
    in_specs=[pl.BlockSpec((EXP_ROWS, S * S), lambda i: (i, 0))],
    out_specs=pl.BlockSpec((EXP_ROWS, S * LANES), lambda i: (i, 0)),
    out_shape=jax.ShapeDtypeStruct((L_PACK, S * LANES), jnp.float32),
)


def _reg_bcast(vec, i):
    return jnp.take_along_axis(vec, jnp.full((LANES,), i, jnp.int32), axis=0,
                               mode="promise_in_bounds")


def _viterbi_tec(p_hbm, rows_hbm, sum_hbm, exp_hbm,
                 idx_v, buf_v, u_v, e_v, res_v, sem0, sem1):
    info = plsc.get_sparse_core_info()
    nc = info.num_cores
    wid = lax.axis_index("s") * nc + lax.axis_index("c")

    @pl.when(wid < B)
    def _run():
        b = wid
        nb = (T_STEPS - 128 * b) // K

        pltpu.sync_copy(rows_hbm.at[b], idx_v)

        u_v[pl.ds(0, LANES)] = jnp.ones((LANES,), jnp.float32)
        u_v[pl.ds(LANES, LANES)] = jnp.ones((LANES,), jnp.float32)
        e_v[...] = jnp.zeros((LANES,), jnp.float32)

        def _gather(blk, slot, sem):
            return pltpu.make_async_copy(
                p_hbm.at[idx_v.at[pl.ds(blk * K, K)]], buf_v.at[slot], sem)

        sems = (sem0, sem1)
        _gather(0, 0, sem0).start()

        def _row_step(r, slot):
            u0v = u_v[pl.ds(0, LANES)]
            u1v = u_v[pl.ds(LANES, LANES)]
            p0 = [jnp.zeros((LANES,), jnp.float32) for _ in range(4)]
            p1 = [jnp.zeros((LANES,), jnp.float32) for _ in range(4)]
            for k in range(S):
                uhalf = u0v if k < LANES else u1v
                uk = _reg_bcast(uhalf, k % LANES)
                w = buf_v[slot, r, pl.ds(k * LANES, LANES)]
                x = plsc.bitcast(w, jnp.bfloat16)
                a0, a1 = plsc.unpack(x, format=plsc.PackFormat.INTERLEAVED)
                p0[k % 4] = p0[k % 4] + a0 * uk
                p1[k % 4] = p1[k % 4] + a1 * uk
            acc0 = (p0[0] + p0[1]) + (p0[2] + p0[3])
            acc1 = (p1[0] + p1[1]) + (p1[2] + p1[3])
            u_v[pl.ds(0, LANES)] = acc0
            u_v[pl.ds(LANES, LANES)] = acc1

            @pl.when((r & 3) == 3)
            def _rescale():
                u0 = u_v[pl.ds(0, LANES)]
                u1 = u_v[pl.ds(LANES, LANES)]
                s = jnp.sum(u0 + u1)
                sv = jnp.full((LANES,), s, jnp.float32)
                e = (plsc.bitcast(sv, jnp.int32) >> 23) - 127
                f = plsc.bitcast((127 - e) << 23, jnp.float32)
                u_v[pl.ds(0, LANES)] = u0 * f
                u_v[pl.ds(LANES, LANES)] = u1 * f
                e_v[...] = e_v[...] + e.astype(jnp.float32)

        def _pair_body(i2, carry):
            for par in (0, 1):
                blk = 2 * i2 + par
                nxt = blk + 1

                @pl.when(nxt < nb)
                def _prefetch():
                    _gather(nxt, 1 - par, sems[1 - par]).start()

                _gather(blk, par, sems[par]).wait()

                def _rows(r, c):
                    _row_step(r, par)
                    return c
                lax.fori_loop(0, K, _rows, 0)
            return carry

        lax.fori_loop(0, nb // 2, _pair_body, 0)

        tot = jnp.sum(u_v[pl.ds(0, LANES)] + u_v[pl.ds(LANES, LANES)])
        res_v[...] = jnp.full((LANES,), tot, jnp.float32)
        pltpu.sync_copy(res_v, sum_hbm.at[b])
        pltpu.sync_copy(e_v, exp_hbm.at[b])


@functools.partial(
    pl.kernel,
    out_type=(jax.ShapeDtypeStruct((B, LANES), jnp.float32),
              jax.ShapeDtypeStruct((B, LANES), jnp.float32)),
    mesh=plsc.VectorSubcoreMesh(core_axis_name="c", subcore_axis_name="s"),
    compiler_params=pltpu.CompilerParams(needs_layout_passes=False),
    scratch_types=[
        pltpu.VMEM((T_STEPS,), jnp.int32),
        pltpu.VMEM((2, K, S * LANES), jnp.float32),
        pltpu.VMEM((S,), jnp.float32),
        pltpu.VMEM((LANES,), jnp.float32),
        pltpu.VMEM((LANES,), jnp.float32),
        pltpu.SemaphoreType.DMA,
        pltpu.SemaphoreType.DMA,
    ],
)
def _viterbi_sc(p_hbm, rows_hbm, sum_hbm, exp_hbm,
                idx_v, buf_v, u_v, e_v, res_v, sem0, sem1):
    _viterbi_tec(p_hbm, rows_hbm, sum_hbm, exp_hbm,
                 idx_v, buf_v, u_v, e_v, res_v, sem0, sem1)


@jax.jit
def kernel(theta_data, batch_sizes):
    theta2d = jnp.transpose(
        theta_data.reshape(L_PACK, 2, LANES, S), (0, 1, 3, 2)
    ).reshape(L_PACK, S * S)
    p_i32 = _exp_tc(theta2d)
    usum, eacc = _viterbi_sc(p_i32, jnp.asarray(_rows_np))
    delta = (jnp.sum(batch_sizes) - L_PACK).astype(jnp.float32)
    ln2 = jnp.float32(np.log(2.0))
    return jnp.log(usum[:, 0]) + eacc[:, 0] * ln2 + delta


# speedup vs baseline: 1.0454x; 1.0019x over previous
import functools

import numpy as np
import jax
import jax.numpy as jnp
from jax import lax
from jax.experimental import pallas as pl
from jax.experimental.pallas import tpu as pltpu
from jax.experimental.pallas import tpu_sc as plsc

T_STEPS = 2048
S = 32
B = 16
LANES = 16
K = 32
L_PACK = 17408
EXP_ROWS = 512

_bs_static = 16 - (np.arange(T_STEPS) // 128)
_off_static = np.concatenate([[0], np.cumsum(_bs_static)])
_rows_np = np.zeros((B, T_STEPS), np.int32)
for _b in range(B):
    _Lb = T_STEPS - 128 * _b
    _rows_np[_b, :_Lb] = _off_static[:_Lb] + _b


def _bf16_bits(x):
    xb = jax.lax.bitcast_convert_type(x, jnp.int32)
    t = xb + 0x7FFF + ((xb >> 16) & 1)
    return (t >> 16) & 0xFFFF


def _exp_body(x_ref, o_ref):
    x = x_ref[...]
    ea = jnp.exp(x[:, :S * LANES])
    eb = jnp.exp(x[:, S * LANES:])
    o_ref[...] = jax.lax.bitcast_convert_type(
        _bf16_bits(ea) | (_bf16_bits(eb) << 16), jnp.float32)


_exp_tc = pl.pallas_call(
    _exp_body,
    grid=(L_PACK // EXP_ROWS,),
    in_specs=[pl.BlockSpec((EXP_ROWS, S * S), lambda i: (i, 0))],
    out_specs=pl.BlockSpec((EXP_ROWS, S * LANES), lambda i: (i, 0)),
    out_shape=jax.ShapeDtypeStruct((L_PACK, S * LANES), jnp.float32),
)


def _reg_bcast(vec, i):
    return jnp.take_along_axis(vec, jnp.full((LANES,), i, jnp.int32), axis=0,
                               mode="promise_in_bounds")


def _viterbi_tec(p_hbm, rows_hbm, sum_hbm, exp_hbm,
                 idx_v, buf_v, u_v, e_v, res_v, sem0, sem1):
    info = plsc.get_sparse_core_info()
    nc = info.num_cores
    wid = lax.axis_index("s") * nc + lax.axis_index("c")

    @pl.when(wid < B)
    def _run():
        b = wid
        nb = (T_STEPS - 128 * b) // K

        pltpu.sync_copy(rows_hbm.at[b], idx_v)

        u_v[pl.ds(0, LANES)] = jnp.ones((LANES,), jnp.float32)
        u_v[pl.ds(LANES, LANES)] = jnp.ones((LANES,), jnp.float32)
        e_v[...] = jnp.zeros((LANES,), jnp.float32)

        def _gather(blk, slot, sem):
            return pltpu.make_async_copy(
                p_hbm.at[idx_v.at[pl.ds(blk * K, K)]], buf_v.at[slot], sem)

        sems = (sem0, sem1)
        _gather(0, 0, sem0).start()

        def _row_step(r, slot):
            u0v = u_v[pl.ds(0, LANES)]
            u1v = u_v[pl.ds(LANES, LANES)]
            p0 = [jnp.zeros((LANES,), jnp.float32) for _ in range(4)]
            p1 = [jnp.zeros((LANES,), jnp.float32) for _ in range(4)]
            for k in range(S):
                uhalf = u0v if k < LANES else u1v
                uk = _reg_bcast(uhalf, k % LANES)
                w = buf_v[slot, r, pl.ds(k * LANES, LANES)]
                x = plsc.bitcast(w, jnp.bfloat16)
                a0, a1 = plsc.unpack(x, format=plsc.PackFormat.INTERLEAVED)
                p0[k % 4] = p0[k % 4] + a0 * uk
                p1[k % 4] = p1[k % 4] + a1 * uk
            acc0 = (p0[0] + p0[1]) + (p0[2] + p0[3])
            acc1 = (p1[0] + p1[1]) + (p1[2] + p1[3])
            u_v[pl.ds(0, LANES)] = acc0
            u_v[pl.ds(LANES, LANES)] = acc1

            @pl.when((r & 3) == 3)
            def _rescale():
                u0 = u_v[pl.ds(0, LANES)]
                u1 = u_v[pl.ds(LANES, LANES)]
                s = jnp.sum(u0 + u1)
                sv = jnp.full((LANES,), s, jnp.float32)
                e = (plsc.bitcast(sv, jnp.int32) >> 23) - 127
                f = plsc.bitcast((127 - e) << 23, jnp.float32)
                u_v[pl.ds(0, LANES)] = u0 * f
                u_v[pl.ds(LANES, LANES)] = u1 * f
                e_v[...] = e_v[...] + e.astype(jnp.float32)

        def _pair_body(i2, carry):
            for par in (0, 1):
                blk = 2 * i2 + par
                nxt = blk + 1

                @pl.when(nxt < nb)
                def _prefetch():
                    _gather(nxt, 1 - par, sems[1 - par]).start()

                _gather(blk, par, sems[par]).wait()

                def _rows(r, c):
                    _row_step(r, par)
                    return c
                lax.fori_loop(0, K, _rows, 0)
            return carry

        lax.fori_loop(0, nb // 2, _pair_body, 0)

        tot = jnp.sum(u_v[pl.ds(0, LANES)] + u_v[pl.ds(LANES, LANES)])
        res_v[...] = jnp.full((LANES,), tot, jnp.float32)
        pltpu.sync_copy(res_v, sum_hbm.at[b])
        pltpu.sync_copy(e_v, exp_hbm.at[b])


@functools.partial(
    pl.kernel,
    out_type=(jax.ShapeDtypeStruct((B, LANES), jnp.float32),
              jax.ShapeDtypeStruct((B, LANES), jnp.float32)),
    mesh=plsc.VectorSubcoreMesh(core_axis_name="c", subcore_axis_name="s"),
    compiler_params=pltpu.CompilerParams(needs_layout_passes=False),
    scratch_types=[
        pltpu.VMEM((T_STEPS,), jnp.int32),
        pltpu.VMEM((2, K, S * LANES), jnp.float32),
        pltpu.VMEM((S,), jnp.float32),
        pltpu.VMEM((LANES,), jnp.float32),
        pltpu.VMEM((LANES,), jnp.float32),
        pltpu.SemaphoreType.DMA,
        pltpu.SemaphoreType.DMA,
    ],
)
def _viterbi_sc(p_hbm, rows_hbm, sum_hbm, exp_hbm,
                idx_v, buf_v, u_v, e_v, res_v, sem0, sem1):
    _viterbi_tec(p_hbm, rows_hbm, sum_hbm, exp_hbm,
                 idx_v, buf_v, u_v, e_v, res_v, sem0, sem1)


@jax.jit
def kernel(theta_data, batch_sizes):
    theta2d = jnp.transpose(
        theta_data.reshape(L_PACK, 2, LANES, S), (0, 1, 3, 2)
    ).reshape(L_PACK, S * S)
    p_i32 = _exp_tc(theta2d)
    usum, eacc = _viterbi_sc(p_i32, jnp.asarray(_rows_np))
    delta = (jnp.sum(batch_sizes) - L_PACK).astype(jnp.float32)
    ln2 = jnp.float32(np.log(2.0))
    return jnp.log(usum[:, 0]) + eacc[:, 0] * ln2 + delta
